# baseline (speedup 1.0000x reference)
import functools

import jax
import jax.numpy as jnp
from jax.experimental import pallas as pl

N = 10000
E = 160000
DIN = 256
H = 4
DOUT = 256

_BM = 512


def _mm_att_kernel(x_ref, w_ref, wa_ref, xw_ref, att_ref):
    xw = jnp.dot(x_ref[...], w_ref[...], preferred_element_type=jnp.float32)
    xw_ref[...] = xw
    att_ref[...] = jnp.dot(xw, wa_ref[...], preferred_element_type=jnp.float32)


def _mm_att(x, w, wa):
    m, k = x.shape
    n = w.shape[1]
    na = wa.shape[1]
    grid = (pl.cdiv(m, _BM),)
    return pl.pallas_call(
        _mm_att_kernel,
        grid=grid,
        in_specs=[
            pl.BlockSpec((_BM, k), lambda i: (i, 0)),
            pl.BlockSpec((k, n), lambda i: (0, 0)),
            pl.BlockSpec((n, na), lambda i: (0, 0)),
        ],
        out_specs=[
            pl.BlockSpec((_BM, n), lambda i: (i, 0)),
            pl.BlockSpec((_BM, na), lambda i: (i, 0)),
        ],
        out_shape=[
            jax.ShapeDtypeStruct((m, n), jnp.float32),
            jax.ShapeDtypeStruct((m, na), jnp.float32),
        ],
    )(x, w, wa)


def _stats_kernel(nrows_total, h_ref, s_ref, q_ref):
    i = pl.program_id(0)

    @pl.when(i == 0)
    def _():
        s_ref[...] = jnp.zeros_like(s_ref)
        q_ref[...] = jnp.zeros_like(q_ref)

    blk = h_ref[...]
    row = i * _BM + jax.lax.broadcasted_iota(jnp.int32, blk.shape, 0)
    blk = jnp.where(row < nrows_total, blk, 0.0)
    s_ref[...] += blk.sum(axis=0, keepdims=True)
    q_ref[...] += (blk * blk).sum(axis=0, keepdims=True)


def _col_stats(h):
    m, n = h.shape
    grid = (pl.cdiv(m, _BM),)
    return pl.pallas_call(
        functools.partial(_stats_kernel, m),
        grid=grid,
        in_specs=[pl.BlockSpec((_BM, n), lambda i: (i, 0))],
        out_specs=[
            pl.BlockSpec((1, n), lambda i: (0, 0)),
            pl.BlockSpec((1, n), lambda i: (0, 0)),
        ],
        out_shape=[
            jax.ShapeDtypeStruct((1, n), jnp.float32),
            jax.ShapeDtypeStruct((1, n), jnp.float32),
        ],
    )(h)


def _norm_mm_kernel(nrows, h_ref, s_ref, q_ref, gnw_ref, gnb_ref, gnms_ref,
                    w_ref, wa_ref, o_ref, att_ref):
    inv_n = 1.0 / nrows
    mean = s_ref[...] * inv_n
    ms = gnms_ref[...]
    var = q_ref[...] * inv_n - (2.0 * ms - ms * ms) * mean * mean
    scale = gnw_ref[...] * jax.lax.rsqrt(var + 1e-5)
    shift = gnb_ref[...] - ms * mean * scale
    y = jnp.maximum(h_ref[...] * scale + shift, 0.0)
    o = jnp.dot(y, w_ref[...], preferred_element_type=jnp.float32)
    o_ref[...] = o
    att_ref[...] = jnp.dot(o, wa_ref[...], preferred_element_type=jnp.float32)


def _norm_relu_mm_att(h, s, q, gn_w, gn_b, gn_ms, w, wa, nrows):
    m, k = h.shape
    n = w.shape[1]
    na = wa.shape[1]
    grid = (pl.cdiv(m, _BM),)
    return pl.pallas_call(
        functools.partial(_norm_mm_kernel, float(nrows)),
        grid=grid,
        in_specs=[
            pl.BlockSpec((_BM, k), lambda i: (i, 0)),
            pl.BlockSpec((1, k), lambda i: (0, 0)),
            pl.BlockSpec((1, k), lambda i: (0, 0)),
            pl.BlockSpec((1, k), lambda i: (0, 0)),
            pl.BlockSpec((1, k), lambda i: (0, 0)),
            pl.BlockSpec((1, k), lambda i: (0, 0)),
            pl.BlockSpec((k, n), lambda i: (0, 0)),
            pl.BlockSpec((n, na), lambda i: (0, 0)),
        ],
        out_specs=[
            pl.BlockSpec((_BM, n), lambda i: (i, 0)),
            pl.BlockSpec((_BM, na), lambda i: (i, 0)),
        ],
        out_shape=[
            jax.ShapeDtypeStruct((m, n), jnp.float32),
            jax.ShapeDtypeStruct((m, na), jnp.float32),
        ],
    )(h, s, q, gn_w.reshape(1, k), gn_b.reshape(1, k), gn_ms.reshape(1, k),
      w, wa)


def _leaky(v):
    return jnp.where(v > 0, v, 0.2 * v)


def _edge_softmax_aggregate(xw, a_s, a_d, src, dst, heads, dout):
    n = xw.shape[0]
    alpha = _leaky(a_s[src] + a_d[dst])
    m = jax.ops.segment_max(alpha, dst, num_segments=n)
    e = jnp.exp(alpha - m[dst])
    s = jax.ops.segment_sum(e, dst, num_segments=n)
    a = e / (s[dst] + 1e-16)
    xw3 = xw.reshape(n, heads, dout)
    out = jax.ops.segment_sum(xw3[src] * a[:, :, None], dst, num_segments=n)
    return out.reshape(n, heads * dout)


def _att_pack(att_src, att_dst):
    h, d = att_src.shape
    eye = jnp.eye(h, dtype=jnp.float32)
    ws = (att_src[:, :, None] * eye[:, None, :]).reshape(h * d, h)
    wd = (att_dst[:, :, None] * eye[:, None, :]).reshape(h * d, h)
    return jnp.concatenate([ws, wd], axis=1)


def kernel(x, edge_index, W1, att_src1, att_dst1, b1, gn_w, gn_b, gn_ms,
           W2, att_src2, att_dst2, b2):
    loop = jnp.arange(N, dtype=edge_index.dtype)
    src = jnp.concatenate([edge_index[0], loop])
    dst = jnp.concatenate([edge_index[1], loop])

    wa1 = _att_pack(att_src1, att_dst1)
    xw1, att1 = _mm_att(x, W1, wa1)
    a_s1, a_d1 = att1[:, :H], att1[:, H:]

    h1 = _edge_softmax_aggregate(xw1, a_s1, a_d1, src, dst, H, DIN)
    h1 = h1 + b1[None, :]

    s, q = _col_stats(h1)
    wa2 = _att_pack(att_src2, att_dst2)
    xw2, att2 = _norm_relu_mm_att(h1, s, q, gn_w, gn_b, gn_ms, W2, wa2, N)
    a_s2, a_d2 = att2[:, :1], att2[:, 1:]

    h2 = _edge_softmax_aggregate(xw2, a_s2, a_d2, src, dst, 1, DOUT)
    return h2 + b2[None, :]


# speedup vs baseline: 6.9525x; 6.9525x over previous
import functools

import jax
import jax.numpy as jnp
from jax import lax
from jax.experimental import pallas as pl
from jax.experimental.pallas import tpu as pltpu
from jax.experimental.pallas import tpu_sc as plsc

N = 10000
E = 160000
DIN = 256
H = 4
DOUT = 256

ET_REAL = E + N
ET_PAD = 180224
NPAD = 10240
PW = 128

_BM = 1000


def _mm1_kernel(x_ref, w_ref, wa_ref, xwp_ref, att_ref):
    p = pl.program_id(1)
    xw = jnp.dot(x_ref[...], w_ref[...], preferred_element_type=jnp.float32)
    xwp_ref[...] = xw[None]

    @pl.when(p == 0)
    def _():
        att_ref[...] = jnp.zeros_like(att_ref)

    att_ref[...] += jnp.dot(xw, wa_ref[...], preferred_element_type=jnp.float32)


def _mm1(x, w, wa):
    k = x.shape[1]
    np_ = w.shape[1] // PW
    grid = (N // _BM, np_)
    return pl.pallas_call(
        _mm1_kernel,
        grid=grid,
        in_specs=[
            pl.BlockSpec((_BM, k), lambda i, p: (i, 0)),
            pl.BlockSpec((k, PW), lambda i, p: (0, p)),
            pl.BlockSpec((PW, 8), lambda i, p: (p, 0)),
        ],
        out_specs=[
            pl.BlockSpec((1, _BM, PW), lambda i, p: (p, i, 0)),
            pl.BlockSpec((_BM, 8), lambda i, p: (i, 0)),
        ],
        out_shape=[
            jax.ShapeDtypeStruct((np_, N, PW), jnp.float32),
            jax.ShapeDtypeStruct((N, 8), jnp.float32),
        ],
    )(x, w, wa)


def _stats_kernel(h_ref, b_ref, s_ref, q_ref):
    i = pl.program_id(1)

    @pl.when(i == 0)
    def _():
        s_ref[...] = jnp.zeros_like(s_ref)
        q_ref[...] = jnp.zeros_like(q_ref)

    blk = h_ref[0] + b_ref[0]
    s_ref[...] += blk.sum(axis=0, keepdims=True)[None]
    q_ref[...] += (blk * blk).sum(axis=0, keepdims=True)[None]


def _col_stats(hp, bp):
    np_ = hp.shape[0]
    grid = (np_, N // _BM)
    return pl.pallas_call(
        _stats_kernel,
        grid=grid,
        in_specs=[
            pl.BlockSpec((1, _BM, PW), lambda p, i: (p, i, 0)),
            pl.BlockSpec((1, 1, PW), lambda p, i: (p, 0, 0)),
        ],
        out_specs=[
            pl.BlockSpec((1, 1, PW), lambda p, i: (p, 0, 0)),
            pl.BlockSpec((1, 1, PW), lambda p, i: (p, 0, 0)),
        ],
        out_shape=[
            jax.ShapeDtypeStruct((np_, 1, PW), jnp.float32),
            jax.ShapeDtypeStruct((np_, 1, PW), jnp.float32),
        ],
    )(hp, bp)


def _norm_mm_kernel(nk, h_ref, s_ref, q_ref, b_ref, gnw_ref, gnb_ref,
                    gnms_ref, w_ref, wa_ref, o_ref, att_ref):
    po = pl.program_id(1)
    k = pl.program_id(2)
    inv_n = 1.0 / N
    mean = s_ref[0] * inv_n
    ms = gnms_ref[0]
    var = q_ref[0] * inv_n - (2.0 * ms - ms * ms) * mean * mean
    scale = gnw_ref[0] * lax.rsqrt(var + 1e-5)
    shift = gnb_ref[0] - ms * mean * scale
    y = jnp.maximum((h_ref[0] + b_ref[0]) * scale + shift, 0.0)

    @pl.when(k == 0)
    def _():
        o_ref[...] = jnp.zeros_like(o_ref)

    o_ref[...] += jnp.dot(y, w_ref[...], preferred_element_type=jnp.float32)[None]

    @pl.when(jnp.logical_and(k == nk - 1, po == 0))
    def _():
        att_ref[...] = jnp.zeros_like(att_ref)

    @pl.when(k == nk - 1)
    def _():
        att_ref[...] += jnp.dot(o_ref[0], wa_ref[...],
                                preferred_element_type=jnp.float32)


def _norm_relu_mm(hp, s, q, bp, gnw, gnb, gnms, w, wa):
    nk = hp.shape[0]
    npo = w.shape[1] // PW
    grid = (N // _BM, npo, nk)
    return pl.pallas_call(
        functools.partial(_norm_mm_kernel, nk),
        grid=grid,
        in_specs=[
            pl.BlockSpec((1, _BM, PW), lambda i, po, k: (k, i, 0)),
            pl.BlockSpec((1, 1, PW), lambda i, po, k: (k, 0, 0)),
            pl.BlockSpec((1, 1, PW), lambda i, po, k: (k, 0, 0)),
            pl.BlockSpec((1, 1, PW), lambda i, po, k: (k, 0, 0)),
            pl.BlockSpec((1, 1, PW), lambda i, po, k: (k, 0, 0)),
            pl.BlockSpec((1, 1, PW), lambda i, po, k: (k, 0, 0)),
            pl.BlockSpec((1, 1, PW), lambda i, po, k: (k, 0, 0)),
            pl.BlockSpec((PW, PW), lambda i, po, k: (k, po)),
            pl.BlockSpec((PW, 8), lambda i, po, k: (po, 0)),
        ],
        out_specs=[
            pl.BlockSpec((1, _BM, PW), lambda i, po, k: (po, i, 0)),
            pl.BlockSpec((_BM, 8), lambda i, po, k: (i, 0)),
        ],
        out_shape=[
            jax.ShapeDtypeStruct((npo, N, PW), jnp.float32),
            jax.ShapeDtypeStruct((N, 8), jnp.float32),
        ],
    )(hp, s, q, bp, gnw, gnb, gnms, w, wa)


def _final_kernel(hp_ref, b_ref, o_ref):
    o_ref[...] = hp_ref[0] + b_ref[...]


def _final_assemble(hp, b):
    np_ = hp.shape[0]
    grid = (N // _BM, np_)
    return pl.pallas_call(
        _final_kernel,
        grid=grid,
        in_specs=[
            pl.BlockSpec((1, _BM, PW), lambda i, p: (p, i, 0)),
            pl.BlockSpec((1, PW), lambda i, p: (0, p)),
        ],
        out_specs=pl.BlockSpec((_BM, PW), lambda i, p: (i, p)),
        out_shape=jax.ShapeDtypeStruct((N, np_ * PW), jnp.float32),
    )(hp, b)


_EW1 = ET_PAD // 16
_CH = 2816
_NSTEP = _CH // 16


def _make_attention_kernel(heads):
    nt = N * heads
    rt16 = ((nt + 2047) // 2048) * 2048
    sl = rt16 // 16
    mesh = plsc.VectorSubcoreMesh(core_axis_name="c", subcore_axis_name="s",
                                  num_cores=1)

    def body(asrc_hbm, adst_hbm, src_hbm, dst_hbm, a_hbm, spart_hbm,
             t1, t2, av, srcv, dstv, mrow, maxall, red_v, acc_v,
             ssum, smax):
        wid = lax.axis_index("s")
        base = wid * _EW1
        zv16 = jnp.zeros((16,), jnp.float32)
        iota16 = lax.iota(jnp.int32, 16)

        pltpu.sync_copy(asrc_hbm, t1.at[pl.ds(0, nt)])
        pltpu.sync_copy(adst_hbm, t2.at[pl.ds(0, nt)])

        mx = jnp.full((16,), -3e38, jnp.float32)
        for ci in range(_EW1 // _CH):
            cbase = base + ci * _CH
            pltpu.sync_copy(src_hbm.at[pl.ds(cbase, _CH)], srcv)
            pltpu.sync_copy(dst_hbm.at[pl.ds(cbase, _CH)], dstv)

            def stepA(j, mx):
                s16 = srcv[pl.ds(j * 16, 16)]
                d16 = dstv[pl.ds(j * 16, 16)]
                for h in range(heads):
                    si = s16 * heads + h
                    di = d16 * heads + h
                    a_s = plsc.load_gather(t1, [si])
                    a_d = plsc.load_gather(t2, [di])
                    z = a_s + a_d
                    al = jnp.where(z > 0, z, 0.2 * z)
                    av[h, pl.ds(j * 16, 16)] = al
                    mx = jnp.maximum(mx, al)
                return mx

            mx = lax.fori_loop(0, _NSTEP, stepA, mx)
            for h in range(heads):
                pltpu.sync_copy(av.at[h], a_hbm.at[h, pl.ds(cbase, _CH)])

        mrow[...] = mx
        pltpu.sync_copy(mrow, smax.at[wid])

        def zt(r, _):
            t1[pl.ds(r * 16, 16)] = zv16
            return 0
        lax.fori_loop(0, rt16 // 16, zt, 0)

        plsc.subcore_barrier()

        pltpu.sync_copy(smax, maxall)
        red = maxall[0, :]
        for r in range(1, 16):
            red = jnp.maximum(red, maxall[r, :])
        mglob = jnp.max(red)
        mv = jnp.full((16,), mglob, jnp.float32)

        for ci in range(4):
            cbase = base + ci * _CH
            pltpu.sync_copy(dst_hbm.at[pl.ds(cbase, _CH)], dstv)
            for h in range(heads):
                pltpu.sync_copy(a_hbm.at[h, pl.ds(cbase, _CH)], av.at[h])

            def stepB(j, _):
                d16 = dstv[pl.ds(j * 16, 16)]
                eid = cbase + j * 16 + iota16
                valid = eid < ET_REAL
                for h in range(heads):
                    al = av[h, pl.ds(j * 16, 16)]
                    e = jnp.where(valid, jnp.exp(al - mv), 0.0)
                    di = d16 * heads + h
                    plsc.addupdate_scatter(t1, [di], e)
                return 0

            lax.fori_loop(0, _NSTEP, stepB, 0)

        pltpu.sync_copy(t1, spart_hbm.at[wid])
        plsc.subcore_barrier()
        sbase = wid * sl

        def za(j, _):
            acc_v[pl.ds(j * 16, 16)] = zv16
            return 0
        lax.fori_loop(0, sl // 16, za, 0)
        for r in range(16):
            pltpu.sync_copy(spart_hbm.at[r, pl.ds(sbase, sl)], red_v)

            def addr(j, _):
                acc_v[pl.ds(j * 16, 16)] += red_v[pl.ds(j * 16, 16)]
                return 0
            lax.fori_loop(0, sl // 16, addr, 0)
        pltpu.sync_copy(acc_v, ssum.at[pl.ds(sbase, sl)])
        plsc.subcore_barrier()
        pltpu.sync_copy(ssum, t2)

        for ci in range(4):
            cbase = base + ci * _CH
            pltpu.sync_copy(dst_hbm.at[pl.ds(cbase, _CH)], dstv)
            for h in range(heads):
                pltpu.sync_copy(a_hbm.at[h, pl.ds(cbase, _CH)], av.at[h])

            def stepC(j, _):
                d16 = dstv[pl.ds(j * 16, 16)]
                eid = cbase + j * 16 + iota16
                valid = eid < ET_REAL
                for h in range(heads):
                    al = av[h, pl.ds(j * 16, 16)]
                    e = jnp.where(valid, jnp.exp(al - mv), 0.0)
                    di = d16 * heads + h
                    s16 = plsc.load_gather(t2, [di])
                    av[h, pl.ds(j * 16, 16)] = e / (s16 + 1e-16)
                return 0

            lax.fori_loop(0, _NSTEP, stepC, 0)
            for h in range(heads):
                pltpu.sync_copy(av.at[h], a_hbm.at[h, pl.ds(cbase, _CH)])

    return pl.kernel(
        body,
        out_type=[
            jax.ShapeDtypeStruct((heads, ET_PAD), jnp.float32),
            jax.ShapeDtypeStruct((16, rt16), jnp.float32),
        ],
        mesh=mesh,
        compiler_params=pltpu.CompilerParams(needs_layout_passes=False),
        scratch_types=[
            pltpu.VMEM((rt16,), jnp.float32),
            pltpu.VMEM((rt16,), jnp.float32),
            pltpu.VMEM((heads, _CH), jnp.float32),
            pltpu.VMEM((_CH,), jnp.int32),
            pltpu.VMEM((_CH,), jnp.int32),
            pltpu.VMEM((16,), jnp.float32),
            pltpu.VMEM((16, 16), jnp.float32),
            pltpu.VMEM((sl,), jnp.float32),
            pltpu.VMEM((sl,), jnp.float32),
            pltpu.VMEM_SHARED((rt16,), jnp.float32),
            pltpu.VMEM_SHARED((16, 16), jnp.float32),
        ],
    )


_EG = 128
_NG = _EW1 // _EG


def _make_aggregate_kernel(num_panels, dout):
    ppc = num_panels // 2
    mesh = plsc.VectorSubcoreMesh(core_axis_name="c", subcore_axis_name="s",
                                  num_cores=2)

    hew = _EW1 // 2
    hg = _NG // 2

    def body(xw_hbm, a_hbm, src_hbm, dst3_hbm, out_hbm,
             acc, rows, src1, a1, dst2, sem):
        c = lax.axis_index("c")
        t = lax.axis_index("s")
        rbase = t * (NPAD // 16)
        zv16 = jnp.zeros((16,), jnp.float32)

        pltpu.sync_copy(dst3_hbm.at[t], dst2)

        def zrows():
            def zr(r, _):
                for j in range(8):
                    rows[r, pl.ds(j * 16, 16)] = zv16
                return 0
            lax.fori_loop(0, _EG, zr, 0)
            for b in range(NPAD // 16 // 128):
                pltpu.sync_copy(rows, acc.at[pl.ds(rbase + b * 128, 128)])

        zrows()
        plsc.subcore_barrier()

        for k in range(ppc):
            p = c * ppc + k
            hp = p * PW // dout
            pn = p * N
            for half in range(2):
                be = t * _EW1 + half * hew
                pltpu.sync_copy(src_hbm.at[pl.ds(be, hew)], src1)
                pltpu.sync_copy(a_hbm.at[pl.ds(hp * ET_PAD + be, hew)], a1)

                def off(j, _):
                    src1[pl.ds(j * 16, 16)] += pn
                    return 0
                lax.fori_loop(0, hew // 16, off, 0)

                def group(g, _):
                    pltpu.async_copy(
                        xw_hbm.at[src1.at[pl.ds(g * _EG, _EG)]],
                        rows, sem).wait()

                    def scale(rr, _):
                        aw16 = a1[pl.ds(g * _EG + rr * 16, 16)]
                        for r in range(16):
                            row = rr * 16 + r
                            av16 = jnp.full((16,), aw16[r], jnp.float32)
                            for j in range(8):
                                rows[row, pl.ds(j * 16, 16)] = (
                                    rows[row, pl.ds(j * 16, 16)] * av16)
                        return 0
                    lax.fori_loop(0, _EG // 16, scale, 0)
                    pltpu.sync_copy(rows,
                                    acc.at[dst2.at[half * hg + g]], add=True)
                    return 0

                lax.fori_loop(0, hg, group, 0)
            plsc.subcore_barrier()
            pltpu.sync_copy(
                acc.at[pl.ds(rbase, NPAD // 16)],
                out_hbm.at[pl.ds(p * NPAD + rbase, NPAD // 16)])
            if k < ppc - 1:
                zrows()
            plsc.subcore_barrier()

    return pl.kernel(
        body,
        out_type=jax.ShapeDtypeStruct((num_panels * NPAD, PW), jnp.float32),
        mesh=mesh,
        compiler_params=pltpu.CompilerParams(needs_layout_passes=False),
        scratch_types=[
            pltpu.VMEM_SHARED((NPAD, PW), jnp.float32),
            pltpu.VMEM((_EG, PW), jnp.float32),
            pltpu.VMEM((hew,), jnp.int32),
            pltpu.VMEM((hew,), jnp.float32),
            pltpu.VMEM((_NG, _EG), jnp.int32),
            pltpu.SemaphoreType.DMA,
        ],
    )


def _att_pack(att_src, att_dst):
    h, d = att_src.shape
    eye = jnp.eye(h, dtype=jnp.float32)
    ws = (att_src[:, :, None] * eye[:, None, :]).reshape(h * d, h)
    wd = (att_dst[:, :, None] * eye[:, None, :]).reshape(h * d, h)
    pad = jnp.zeros((h * d, 8 - 2 * h), jnp.float32)
    return jnp.concatenate([ws, wd, pad], axis=1)


def kernel(x, edge_index, W1, att_src1, att_dst1, b1, gn_w, gn_b, gn_ms,
           W2, att_src2, att_dst2, b2):
    loop = jnp.arange(N, dtype=jnp.int32)
    padz = jnp.zeros((ET_PAD - ET_REAL,), jnp.int32)
    src = jnp.concatenate([edge_index[0].astype(jnp.int32), loop, padz])
    dst = jnp.concatenate([edge_index[1].astype(jnp.int32), loop, padz])
    dst3 = dst.reshape(16, _NG, _EG)

    wa1 = _att_pack(att_src1, att_dst1)
    xw1p, att1 = _mm1(x, W1, wa1)
    asrc1 = att1[:, :H].reshape(N * H)
    adst1 = att1[:, H:2 * H].reshape(N * H)

    a1w, _ = _make_attention_kernel(H)(asrc1, adst1, src, dst)
    out1 = _make_aggregate_kernel(H * DIN // PW, DIN)(
        xw1p.reshape(H * DIN // PW * N, PW), a1w.reshape(-1), src, dst3)
    h1p = out1.reshape(H * DIN // PW, NPAD, PW)

    b1p = b1.reshape(H * DIN // PW, 1, PW)
    s, q = _col_stats(h1p, b1p)
    wa2 = _att_pack(att_src2, att_dst2)
    xw2p, att2 = _norm_relu_mm(
        h1p, s, q, b1p,
        gn_w.reshape(H * DIN // PW, 1, PW), gn_b.reshape(H * DIN // PW, 1, PW),
        gn_ms.reshape(H * DIN // PW, 1, PW), W2, wa2)
    asrc2 = att2[:, 0].reshape(N)
    adst2 = att2[:, 1].reshape(N)

    a2w, _ = _make_attention_kernel(1)(asrc2, adst2, src, dst)
    out2 = _make_aggregate_kernel(DOUT // PW, DOUT)(
        xw2p.reshape(DOUT // PW * N, PW), a2w.reshape(-1), src, dst3)
    h2p = out2.reshape(DOUT // PW, NPAD, PW)

    return _final_assemble(h2p, b2.reshape(1, DOUT))


# speedup vs baseline: 7.4200x; 1.0673x over previous
import functools

import jax
import jax.numpy as jnp
from jax import lax
from jax.experimental import pallas as pl
from jax.experimental.pallas import tpu as pltpu
from jax.experimental.pallas import tpu_sc as plsc

N = 10000
E = 160000
DIN = 256
H = 4
DOUT = 256

ET_REAL = E + N
ET_PAD = 180224
NPAD = 10112
PW = 128

_BM = 1000


def _mm1_kernel(x_ref, w_ref, wa_ref, xwp_ref, att_ref):
    p = pl.program_id(1)
    xw = jnp.dot(x_ref[...], w_ref[...], preferred_element_type=jnp.float32)
    xwp_ref[...] = xw[None]

    @pl.when(p == 0)
    def _():
        att_ref[...] = jnp.zeros_like(att_ref)

    att_ref[...] += jnp.dot(xw, wa_ref[...], preferred_element_type=jnp.float32)


def _mm1(x, w, wa):
    k = x.shape[1]
    np_ = w.shape[1] // PW
    grid = (N // _BM, np_)
    return pl.pallas_call(
        _mm1_kernel,
        grid=grid,
        in_specs=[
            pl.BlockSpec((_BM, k), lambda i, p: (i, 0)),
            pl.BlockSpec((k, PW), lambda i, p: (0, p)),
            pl.BlockSpec((PW, 8), lambda i, p: (p, 0)),
        ],
        out_specs=[
            pl.BlockSpec((1, _BM, PW), lambda i, p: (p, i, 0)),
            pl.BlockSpec((_BM, 8), lambda i, p: (i, 0)),
        ],
        out_shape=[
            jax.ShapeDtypeStruct((np_, N, PW), jnp.float32),
            jax.ShapeDtypeStruct((N, 8), jnp.float32),
        ],
    )(x, w, wa)


def _stats_kernel(h_ref, b_ref, s_ref, q_ref):
    i = pl.program_id(1)

    @pl.when(i == 0)
    def _():
        s_ref[...] = jnp.zeros_like(s_ref)
        q_ref[...] = jnp.zeros_like(q_ref)

    blk = h_ref[0] + b_ref[0]
    s_ref[...] += blk.sum(axis=0, keepdims=True)[None]
    q_ref[...] += (blk * blk).sum(axis=0, keepdims=True)[None]


def _col_stats(hp, bp):
    np_ = hp.shape[0]
    grid = (np_, N // _BM)
    return pl.pallas_call(
        _stats_kernel,
        grid=grid,
        in_specs=[
            pl.BlockSpec((1, _BM, PW), lambda p, i: (p, i, 0)),
            pl.BlockSpec((1, 1, PW), lambda p, i: (p, 0, 0)),
        ],
        out_specs=[
            pl.BlockSpec((1, 1, PW), lambda p, i: (p, 0, 0)),
            pl.BlockSpec((1, 1, PW), lambda p, i: (p, 0, 0)),
        ],
        out_shape=[
            jax.ShapeDtypeStruct((np_, 1, PW), jnp.float32),
            jax.ShapeDtypeStruct((np_, 1, PW), jnp.float32),
        ],
    )(hp, bp)


def _norm_mm_kernel(nk, h_ref, s_ref, q_ref, b_ref, gnw_ref, gnb_ref,
                    gnms_ref, w_ref, wa_ref, o_ref, att_ref):
    po = pl.program_id(1)
    k = pl.program_id(2)
    inv_n = 1.0 / N
    mean = s_ref[0] * inv_n
    ms = gnms_ref[0]
    var = q_ref[0] * inv_n - (2.0 * ms - ms * ms) * mean * mean
    scale = gnw_ref[0] * lax.rsqrt(var + 1e-5)
    shift = gnb_ref[0] - ms * mean * scale
    y = jnp.maximum((h_ref[0] + b_ref[0]) * scale + shift, 0.0)

    @pl.when(k == 0)
    def _():
        o_ref[...] = jnp.zeros_like(o_ref)

    o_ref[...] += jnp.dot(y, w_ref[...], preferred_element_type=jnp.float32)[None]

    @pl.when(jnp.logical_and(k == nk - 1, po == 0))
    def _():
        att_ref[...] = jnp.zeros_like(att_ref)

    @pl.when(k == nk - 1)
    def _():
        att_ref[...] += jnp.dot(o_ref[0], wa_ref[...],
                                preferred_element_type=jnp.float32)


def _norm_relu_mm(hp, s, q, bp, gnw, gnb, gnms, w, wa):
    nk = hp.shape[0]
    npo = w.shape[1] // PW
    grid = (N // _BM, npo, nk)
    return pl.pallas_call(
        functools.partial(_norm_mm_kernel, nk),
        grid=grid,
        in_specs=[
            pl.BlockSpec((1, _BM, PW), lambda i, po, k: (k, i, 0)),
            pl.BlockSpec((1, 1, PW), lambda i, po, k: (k, 0, 0)),
            pl.BlockSpec((1, 1, PW), lambda i, po, k: (k, 0, 0)),
            pl.BlockSpec((1, 1, PW), lambda i, po, k: (k, 0, 0)),
            pl.BlockSpec((1, 1, PW), lambda i, po, k: (k, 0, 0)),
            pl.BlockSpec((1, 1, PW), lambda i, po, k: (k, 0, 0)),
            pl.BlockSpec((1, 1, PW), lambda i, po, k: (k, 0, 0)),
            pl.BlockSpec((PW, PW), lambda i, po, k: (k, po)),
            pl.BlockSpec((PW, 8), lambda i, po, k: (po, 0)),
        ],
        out_specs=[
            pl.BlockSpec((1, _BM, PW), lambda i, po, k: (po, i, 0)),
            pl.BlockSpec((_BM, 8), lambda i, po, k: (i, 0)),
        ],
        out_shape=[
            jax.ShapeDtypeStruct((npo, N, PW), jnp.float32),
            jax.ShapeDtypeStruct((N, 8), jnp.float32),
        ],
    )(hp, s, q, bp, gnw, gnb, gnms, w, wa)


def _final_kernel(hp_ref, b_ref, o_ref):
    o_ref[...] = hp_ref[0] + b_ref[...]


def _final_assemble(hp, b):
    np_ = hp.shape[0]
    grid = (N // _BM, np_)
    return pl.pallas_call(
        _final_kernel,
        grid=grid,
        in_specs=[
            pl.BlockSpec((1, _BM, PW), lambda i, p: (p, i, 0)),
            pl.BlockSpec((1, PW), lambda i, p: (0, p)),
        ],
        out_specs=pl.BlockSpec((_BM, PW), lambda i, p: (i, p)),
        out_shape=jax.ShapeDtypeStruct((N, np_ * PW), jnp.float32),
    )(hp, b)


_EW1 = ET_PAD // 16
_CH = 2816
_NSTEP = _CH // 16


def _make_attention_kernel(heads):
    nt = N * heads
    rt16 = ((nt + 2047) // 2048) * 2048
    sl = rt16 // 16
    mesh = plsc.VectorSubcoreMesh(core_axis_name="c", subcore_axis_name="s",
                                  num_cores=1)

    def body(asrc_hbm, adst_hbm, src_hbm, dst_hbm, a_hbm, spart_hbm,
             t1, t2, av, srcv, dstv, mrow, maxall, red_v, acc_v,
             ssum, smax):
        wid = lax.axis_index("s")
        base = wid * _EW1
        zv16 = jnp.zeros((16,), jnp.float32)
        iota16 = lax.iota(jnp.int32, 16)

        pltpu.sync_copy(asrc_hbm, t1.at[pl.ds(0, nt)])
        pltpu.sync_copy(adst_hbm, t2.at[pl.ds(0, nt)])

        mx = jnp.full((16,), -3e38, jnp.float32)
        for ci in range(_EW1 // _CH):
            cbase = base + ci * _CH
            pltpu.sync_copy(src_hbm.at[pl.ds(cbase, _CH)], srcv)
            pltpu.sync_copy(dst_hbm.at[pl.ds(cbase, _CH)], dstv)

            def stepA(j, mx):
                s16 = srcv[pl.ds(j * 16, 16)]
                d16 = dstv[pl.ds(j * 16, 16)]
                for h in range(heads):
                    si = s16 * heads + h
                    di = d16 * heads + h
                    a_s = plsc.load_gather(t1, [si])
                    a_d = plsc.load_gather(t2, [di])
                    z = a_s + a_d
                    al = jnp.where(z > 0, z, 0.2 * z)
                    av[h, pl.ds(j * 16, 16)] = al
                    mx = jnp.maximum(mx, al)
                return mx

            mx = lax.fori_loop(0, _NSTEP, stepA, mx)
            for h in range(heads):
                pltpu.sync_copy(av.at[h], a_hbm.at[h, pl.ds(cbase, _CH)])

        mrow[...] = mx
        pltpu.sync_copy(mrow, smax.at[wid])

        def zt(r, _):
            t1[pl.ds(r * 16, 16)] = zv16
            return 0
        lax.fori_loop(0, rt16 // 16, zt, 0)

        plsc.subcore_barrier()

        pltpu.sync_copy(smax, maxall)
        red = maxall[0, :]
        for r in range(1, 16):
            red = jnp.maximum(red, maxall[r, :])
        mglob = jnp.max(red)
        mv = jnp.full((16,), mglob, jnp.float32)

        for ci in range(4):
            cbase = base + ci * _CH
            pltpu.sync_copy(dst_hbm.at[pl.ds(cbase, _CH)], dstv)
            for h in range(heads):
                pltpu.sync_copy(a_hbm.at[h, pl.ds(cbase, _CH)], av.at[h])

            def stepB(j, _):
                d16 = dstv[pl.ds(j * 16, 16)]
                eid = cbase + j * 16 + iota16
                valid = eid < ET_REAL
                for h in range(heads):
                    al = av[h, pl.ds(j * 16, 16)]
                    e = jnp.where(valid, jnp.exp(al - mv), 0.0)
                    di = d16 * heads + h
                    plsc.addupdate_scatter(t1, [di], e)
                return 0

            lax.fori_loop(0, _NSTEP, stepB, 0)

        pltpu.sync_copy(t1, spart_hbm.at[wid])
        plsc.subcore_barrier()
        sbase = wid * sl

        def za(j, _):
            acc_v[pl.ds(j * 16, 16)] = zv16
            return 0
        lax.fori_loop(0, sl // 16, za, 0)
        for r in range(16):
            pltpu.sync_copy(spart_hbm.at[r, pl.ds(sbase, sl)], red_v)

            def addr(j, _):
                acc_v[pl.ds(j * 16, 16)] += red_v[pl.ds(j * 16, 16)]
                return 0
            lax.fori_loop(0, sl // 16, addr, 0)
        pltpu.sync_copy(acc_v, ssum.at[pl.ds(sbase, sl)])
        plsc.subcore_barrier()
        pltpu.sync_copy(ssum, t2)

        for ci in range(4):
            cbase = base + ci * _CH
            pltpu.sync_copy(dst_hbm.at[pl.ds(cbase, _CH)], dstv)
            for h in range(heads):
                pltpu.sync_copy(a_hbm.at[h, pl.ds(cbase, _CH)], av.at[h])

            def stepC(j, _):
                d16 = dstv[pl.ds(j * 16, 16)]
                eid = cbase + j * 16 + iota16
                valid = eid < ET_REAL
                for h in range(heads):
                    al = av[h, pl.ds(j * 16, 16)]
                    e = jnp.where(valid, jnp.exp(al - mv), 0.0)
                    di = d16 * heads + h
                    s16 = plsc.load_gather(t2, [di])
                    av[h, pl.ds(j * 16, 16)] = e / (s16 + 1e-16)
                return 0

            lax.fori_loop(0, _NSTEP, stepC, 0)
            for h in range(heads):
                pltpu.sync_copy(av.at[h], a_hbm.at[h, pl.ds(cbase, _CH)])

    return pl.kernel(
        body,
        out_type=[
            jax.ShapeDtypeStruct((heads, ET_PAD), jnp.float32),
            jax.ShapeDtypeStruct((16, rt16), jnp.float32),
        ],
        mesh=mesh,
        compiler_params=pltpu.CompilerParams(needs_layout_passes=False),
        scratch_types=[
            pltpu.VMEM((rt16,), jnp.float32),
            pltpu.VMEM((rt16,), jnp.float32),
            pltpu.VMEM((heads, _CH), jnp.float32),
            pltpu.VMEM((_CH,), jnp.int32),
            pltpu.VMEM((_CH,), jnp.int32),
            pltpu.VMEM((16,), jnp.float32),
            pltpu.VMEM((16, 16), jnp.float32),
            pltpu.VMEM((sl,), jnp.float32),
            pltpu.VMEM((sl,), jnp.float32),
            pltpu.VMEM_SHARED((rt16,), jnp.float32),
            pltpu.VMEM_SHARED((16, 16), jnp.float32),
        ],
    )


_EG = 128
_NG = _EW1 // _EG


def _make_aggregate_kernel(num_panels, dout):
    ppc = num_panels // 2
    mesh = plsc.VectorSubcoreMesh(core_axis_name="c", subcore_axis_name="s",
                                  num_cores=2)

    qew = _EW1 // 4
    qg = _NG // 4
    rpt = NPAD // 16

    def body(xw_hbm, a_hbm, src_hbm, dst3_hbm, out_hbm,
             acc, rows, src1, a1, dst2, sem):
        c = lax.axis_index("c")
        t = lax.axis_index("s")
        rbase = t * rpt
        zv16 = jnp.zeros((16,), jnp.float32)

        pltpu.sync_copy(dst3_hbm.at[t], dst2)

        def zrows():
            def zr(r, _):
                for j in range(8):
                    rows[0, r, pl.ds(j * 16, 16)] = zv16
                return 0
            lax.fori_loop(0, _EG, zr, 0)
            for b in range(rpt // 128):
                pltpu.sync_copy(rows.at[0],
                                acc.at[pl.ds(rbase + b * 128, 128)])
            rem = rpt % 128
            if rem:
                pltpu.sync_copy(
                    rows.at[0, pl.ds(0, rem)],
                    acc.at[pl.ds(rbase + (rpt // 128) * 128, rem)])

        zrows()
        plsc.subcore_barrier()

        for k in range(ppc):
            p = c * ppc + k
            hp = p * PW // dout
            pn = p * N
            for quarter in range(4):
                be = t * _EW1 + quarter * qew
                pltpu.sync_copy(src_hbm.at[pl.ds(be, qew)], src1)
                pltpu.sync_copy(a_hbm.at[pl.ds(hp * ET_PAD + be, qew)], a1)

                def off(j, _):
                    src1[pl.ds(j * 16, 16)] += pn
                    return 0
                lax.fori_loop(0, qew // 16, off, 0)

                pltpu.async_copy(
                    xw_hbm.at[src1.at[pl.ds(0, _EG)]],
                    rows.at[0], sem.at[0])

                def group(g, _):
                    q = lax.bitwise_and(g, 1)
                    qn = 1 - q

                    @pl.when(g + 1 < qg)
                    def _():
                        pltpu.async_copy(
                            xw_hbm.at[src1.at[pl.ds((g + 1) * _EG, _EG)]],
                            rows.at[qn], sem.at[qn])

                    pltpu.make_async_copy(
                        xw_hbm.at[src1.at[pl.ds(g * _EG, _EG)]],
                        rows.at[q], sem.at[q]).wait()

                    def scale(rr, _):
                        aw16 = a1[pl.ds(g * _EG + rr * 16, 16)]
                        for r in range(16):
                            row = rr * 16 + r
                            av16 = jnp.full((16,), aw16[r], jnp.float32)
                            for j in range(8):
                                rows[q, row, pl.ds(j * 16, 16)] = (
                                    rows[q, row, pl.ds(j * 16, 16)] * av16)
                        return 0
                    lax.fori_loop(0, _EG // 16, scale, 0)
                    pltpu.sync_copy(rows.at[q],
                                    acc.at[dst2.at[quarter * qg + g]],
                                    add=True)
                    return 0

                lax.fori_loop(0, qg, group, 0)
            plsc.subcore_barrier()
            pltpu.sync_copy(
                acc.at[pl.ds(rbase, rpt)],
                out_hbm.at[pl.ds(p * NPAD + rbase, rpt)])
            if k < ppc - 1:
                zrows()
            plsc.subcore_barrier()

    return pl.kernel(
        body,
        out_type=jax.ShapeDtypeStruct((num_panels * NPAD, PW), jnp.float32),
        mesh=mesh,
        compiler_params=pltpu.CompilerParams(needs_layout_passes=False),
        scratch_types=[
            pltpu.VMEM_SHARED((NPAD, PW), jnp.float32),
            pltpu.VMEM((2, _EG, PW), jnp.float32),
            pltpu.VMEM((qew,), jnp.int32),
            pltpu.VMEM((qew,), jnp.float32),
            pltpu.VMEM((_NG, _EG), jnp.int32),
            pltpu.SemaphoreType.DMA((2,)),
        ],
    )


def _att_pack(att_src, att_dst):
    h, d = att_src.shape
    eye = jnp.eye(h, dtype=jnp.float32)
    ws = (att_src[:, :, None] * eye[:, None, :]).reshape(h * d, h)
    wd = (att_dst[:, :, None] * eye[:, None, :]).reshape(h * d, h)
    pad = jnp.zeros((h * d, 8 - 2 * h), jnp.float32)
    return jnp.concatenate([ws, wd, pad], axis=1)


def kernel(x, edge_index, W1, att_src1, att_dst1, b1, gn_w, gn_b, gn_ms,
           W2, att_src2, att_dst2, b2):
    loop = jnp.arange(N, dtype=jnp.int32)
    padz = jnp.zeros((ET_PAD - ET_REAL,), jnp.int32)
    src = jnp.concatenate([edge_index[0].astype(jnp.int32), loop, padz])
    dst = jnp.concatenate([edge_index[1].astype(jnp.int32), loop, padz])
    dst3 = dst.reshape(16, _NG, _EG)

    wa1 = _att_pack(att_src1, att_dst1)
    xw1p, att1 = _mm1(x, W1, wa1)
    asrc1 = att1[:, :H].reshape(N * H)
    adst1 = att1[:, H:2 * H].reshape(N * H)

    a1w, _ = _make_attention_kernel(H)(asrc1, adst1, src, dst)
    out1 = _make_aggregate_kernel(H * DIN // PW, DIN)(
        xw1p.reshape(H * DIN // PW * N, PW), a1w.reshape(-1), src, dst3)
    h1p = out1.reshape(H * DIN // PW, NPAD, PW)

    b1p = b1.reshape(H * DIN // PW, 1, PW)
    s, q = _col_stats(h1p, b1p)
    wa2 = _att_pack(att_src2, att_dst2)
    xw2p, att2 = _norm_relu_mm(
        h1p, s, q, b1p,
        gn_w.reshape(H * DIN // PW, 1, PW), gn_b.reshape(H * DIN // PW, 1, PW),
        gn_ms.reshape(H * DIN // PW, 1, PW), W2, wa2)
    asrc2 = att2[:, 0].reshape(N)
    adst2 = att2[:, 1].reshape(N)

    a2w, _ = _make_attention_kernel(1)(asrc2, adst2, src, dst)
    out2 = _make_aggregate_kernel(DOUT // PW, DOUT)(
        xw2p.reshape(DOUT // PW * N, PW), a2w.reshape(-1), src, dst3)
    h2p = out2.reshape(DOUT // PW, NPAD, PW)

    return _final_assemble(h2p, b2.reshape(1, DOUT))
